# baseline (speedup 1.0000x reference)
import jax
import jax.numpy as jnp
from jax import lax
from jax.experimental import pallas as pl
from jax.experimental.pallas import tpu as pltpu
from jax.experimental.pallas import tpu_sc as plsc

_N = 10000
_E = 320000
_D_IN = 128
_D_HID = 32
_D_OUT = 2
_W2P = 16
_EPS = 1e-5

_NC = 2
_NS = 16
_GE = 512
_TG = 640
_GF = _TG // _NS
_NBUF = 2
_E_PAD = _TG * _GE
_ROWS = 10112
_RPT = _ROWS // _NS
_DW = 8


def _sc_agg(width, with_deg):
  mesh = plsc.VectorSubcoreMesh(core_axis_name="c", subcore_axis_name="s",
                                num_cores=_NC, num_subcores=_NS)
  out_type = [jax.ShapeDtypeStruct((_ROWS, width), jnp.float32)]
  scratch = [
      pltpu.VMEM((_GF, _GE), jnp.int32),
      pltpu.VMEM((_GF, _GE), jnp.int32),
      pltpu.VMEM((_NBUF, _GE, width), jnp.float32),
      pltpu.VMEM_SHARED((_ROWS, width), jnp.float32),
  ]
  if with_deg:
    out_type.append(jax.ShapeDtypeStruct((_ROWS, _DW), jnp.float32))
    scratch += [
        pltpu.VMEM((_GE, _DW), jnp.float32),
        pltpu.VMEM_SHARED((_ROWS, _DW), jnp.float32),
    ]
  scratch += [pltpu.SemaphoreType.DMA] * (3 * _NBUF)

  def body(*refs):
    if with_deg:
      (y_hbm, src_hbm, dst_hbm, zf_hbm, zd_hbm, ones_hbm, acc_out, deg_out,
       src_v, dst_v, msgs_v, acc_sh, ones_v, deg_sh, *sems) = refs
    else:
      (y_hbm, src_hbm, dst_hbm, zf_hbm, acc_out,
       src_v, dst_v, msgs_v, acc_sh, *sems) = refs
    gsem = sems[:_NBUF]
    ssem = sems[_NBUF:2 * _NBUF]
    dsem = sems[2 * _NBUF:]

    c = lax.axis_index("c")
    s = lax.axis_index("s")
    row0 = s * _RPT
    lo = s * _GF

    @pl.when(c == 0)
    def _features():
      pltpu.sync_copy(zf_hbm, acc_sh.at[pl.ds(row0, _RPT)])
      pltpu.sync_copy(src_hbm.at[pl.ds(lo, _GF)], src_v)
      pltpu.sync_copy(dst_hbm.at[pl.ds(lo, _GF)], dst_v)
      for b in range(_NBUF):
        pltpu.async_copy(y_hbm.at[src_v.at[b]], msgs_v.at[b], gsem[b])
      plsc.subcore_barrier()

      def step(i, carry):
        for b in range(_NBUF):
          r = i * _NBUF + b
          pltpu.make_async_copy(y_hbm.at[src_v.at[r]], msgs_v.at[b],
                                gsem[b]).wait()
          pltpu.async_copy(msgs_v.at[b], acc_sh.at[dst_v.at[r]], ssem[b],
                           add=True).wait()

          @pl.when(r + _NBUF < _GF)
          def _start_next():
            pltpu.async_copy(y_hbm.at[src_v.at[r + _NBUF]], msgs_v.at[b],
                             gsem[b])
        return carry

      lax.fori_loop(0, _GF // _NBUF, step, 0)
      plsc.subcore_barrier()
      pltpu.sync_copy(acc_sh.at[pl.ds(row0, _RPT)],
                      acc_out.at[pl.ds(row0, _RPT)])

    if with_deg:
      @pl.when(c == 1)
      def _degrees():
        pltpu.sync_copy(zd_hbm, deg_sh.at[pl.ds(row0, _RPT)])
        pltpu.sync_copy(ones_hbm, ones_v)
        pltpu.sync_copy(dst_hbm.at[pl.ds(lo, _GF)], dst_v)
        plsc.subcore_barrier()

        def dstep(i, carry):
          for b in range(_NBUF):
            r = i * _NBUF + b

            @pl.when(r >= _NBUF)
            def _drain():
              pltpu.make_async_copy(ones_v, deg_sh.at[dst_v.at[r]],
                                    dsem[b]).wait()

            pltpu.async_copy(ones_v, deg_sh.at[dst_v.at[r]], dsem[b],
                             add=True)
          return carry

        lax.fori_loop(0, _GF // _NBUF, dstep, 0)
        for b in range(_NBUF):
          pltpu.make_async_copy(ones_v, deg_sh.at[dst_v.at[0]],
                                dsem[b]).wait()
        plsc.subcore_barrier()
        pltpu.sync_copy(deg_sh.at[pl.ds(row0, _RPT)],
                        deg_out.at[pl.ds(row0, _RPT)])

  return pl.kernel(
      body, out_type=out_type, mesh=mesh, scratch_types=scratch,
      compiler_params=pltpu.CompilerParams(use_tc_tiling_on_sc=False))


def _stage_a(x, W1_l, W1_r, b1):
  def body(x_ref, wl_ref, wr_ref, b_ref, y_ref, z_ref):
    xv = x_ref[...]
    y_ref[...] = jnp.dot(xv, wl_ref[...], preferred_element_type=jnp.float32)
    z_ref[...] = (jnp.dot(xv, wr_ref[...], preferred_element_type=jnp.float32)
                  + b_ref[...])

  return pl.pallas_call(
      body,
      out_shape=[jax.ShapeDtypeStruct((_N, _D_HID), jnp.float32),
                 jax.ShapeDtypeStruct((_N, _D_HID), jnp.float32)],
  )(x, W1_l, W1_r, b1)


def _stage_b(acc1, deg, z1, gamma, beta, W2lp, W2_r, b2):
  def body(acc_ref, deg_ref, z1_ref, g_ref, be_ref, wl_ref, wr_ref, b2_ref,
           y2_ref, z2_ref):
    invd = 1.0 / jnp.maximum(deg_ref[:_N, :1], 1.0)
    pre = acc_ref[:_N, :] * invd + z1_ref[...]
    mu = jnp.mean(pre, axis=0, keepdims=True)
    var = jnp.mean((pre - mu) ** 2, axis=0, keepdims=True)
    h = (pre - mu) * lax.rsqrt(var + _EPS) * g_ref[...] + be_ref[...]
    h = jnp.maximum(h, 0.0)
    y2_ref[...] = jnp.dot(h, wl_ref[...], preferred_element_type=jnp.float32)
    z2_ref[...] = (jnp.dot(h, wr_ref[...], preferred_element_type=jnp.float32)
                   + b2_ref[...])

  return pl.pallas_call(
      body,
      out_shape=[jax.ShapeDtypeStruct((_N, _W2P), jnp.float32),
                 jax.ShapeDtypeStruct((_N, _D_OUT), jnp.float32)],
  )(acc1, deg, z1, gamma, beta, W2lp, W2_r, b2)


def _stage_c(acc2, deg, z2, gamma, beta):
  def body(acc_ref, deg_ref, z2_ref, g_ref, be_ref, out_ref):
    invd = 1.0 / jnp.maximum(deg_ref[:_N, :1], 1.0)
    pre = acc_ref[:_N, :_D_OUT] * invd + z2_ref[...]
    mu = jnp.mean(pre, axis=0, keepdims=True)
    var = jnp.mean((pre - mu) ** 2, axis=0, keepdims=True)
    h = (pre - mu) * lax.rsqrt(var + _EPS) * g_ref[...] + be_ref[...]
    m = jnp.max(h, axis=1, keepdims=True)
    lse = jnp.log(jnp.sum(jnp.exp(h - m), axis=1, keepdims=True)) + m
    out_ref[...] = h - lse

  return pl.pallas_call(
      body,
      out_shape=jax.ShapeDtypeStruct((_N, _D_OUT), jnp.float32),
  )(acc2, deg, z2, gamma, beta)


def kernel(x, edge_index, W1_l, W1_r, b1, bn1_gamma, bn1_beta,
           W2_l, W2_r, b2, bn2_gamma, bn2_beta):
  src = edge_index[0]
  dst = edge_index[1]
  pad = _E_PAD - _E
  src3 = jnp.concatenate([src, jnp.zeros((pad,), jnp.int32)]).reshape(
      _TG, _GE)
  dst_pad = _N + jnp.arange(pad, dtype=jnp.int32) % (_ROWS - _N)
  dst3 = jnp.concatenate([dst, dst_pad]).reshape(_TG, _GE)
  zf32 = jnp.zeros((_RPT, _D_HID), jnp.float32)
  zf16 = jnp.zeros((_RPT, _W2P), jnp.float32)
  zd = jnp.zeros((_RPT, _DW), jnp.float32)
  ones = jnp.ones((_GE, _DW), jnp.float32)
  W2lp = jnp.pad(W2_l, ((0, 0), (0, _W2P - _D_OUT)))

  y1, z1 = _stage_a(x, W1_l, W1_r, b1)
  acc1, deg = _sc_agg(_D_HID, True)(y1, src3, dst3, zf32, zd, ones)
  y2p, z2 = _stage_b(acc1, deg, z1, bn1_gamma, bn1_beta, W2lp, W2_r, b2)
  (acc2,) = _sc_agg(_W2P, False)(y2p, src3, dst3, zf16)
  return _stage_c(acc2, deg, z2, bn2_gamma, bn2_beta)


# speedup vs baseline: 1.6368x; 1.6368x over previous
import jax
import jax.numpy as jnp
from jax import lax
from jax.experimental import pallas as pl
from jax.experimental.pallas import tpu as pltpu
from jax.experimental.pallas import tpu_sc as plsc

_N = 10000
_E = 320000
_D_IN = 128
_D_HID = 32
_D_OUT = 2
_W2P = 16
_EPS = 1e-5

_NC = 2
_NS = 16
_GE = 500
_TG = 640
_GF = _TG // _NS
_NBUF = 2
_ROWS = 10112
_RPT = _ROWS // _NS
_DW = 8


def _sc_agg(width, with_deg):
  mesh = plsc.VectorSubcoreMesh(core_axis_name="c", subcore_axis_name="s",
                                num_cores=_NC, num_subcores=_NS)
  out_type = [jax.ShapeDtypeStruct((_ROWS, width), jnp.float32)]
  scratch = [
      pltpu.VMEM((_GF, _GE), jnp.int32),
      pltpu.VMEM((_GF, _GE), jnp.int32),
      pltpu.VMEM((_NBUF, _GE, width), jnp.float32),
      pltpu.VMEM_SHARED((_ROWS, width), jnp.float32),
  ]
  if with_deg:
    out_type.append(jax.ShapeDtypeStruct((_ROWS, _DW), jnp.float32))
    scratch += [
        pltpu.VMEM((_GE, _DW), jnp.float32),
        pltpu.VMEM_SHARED((_ROWS, _DW), jnp.float32),
    ]
  scratch += [pltpu.SemaphoreType.DMA] * (3 * _NBUF)

  def body(*refs):
    if with_deg:
      (y_hbm, src_hbm, dst_hbm, zf_hbm, zd_hbm, ones_hbm, acc_out, deg_out,
       src_v, dst_v, msgs_v, acc_sh, ones_v, deg_sh, *sems) = refs
    else:
      (y_hbm, src_hbm, dst_hbm, zf_hbm, acc_out,
       src_v, dst_v, msgs_v, acc_sh, *sems) = refs
    gsem = sems[:_NBUF]
    ssem = sems[_NBUF:2 * _NBUF]
    dsem = sems[2 * _NBUF:]

    c = lax.axis_index("c")
    s = lax.axis_index("s")
    row0 = s * _RPT
    lo = s * _GF

    @pl.when(c == 0)
    def _features():
      pltpu.sync_copy(zf_hbm, acc_sh.at[pl.ds(row0, _RPT)])
      pltpu.sync_copy(src_hbm.at[pl.ds(lo, _GF)], src_v)
      pltpu.sync_copy(dst_hbm.at[pl.ds(lo, _GF)], dst_v)
      for b in range(_NBUF):
        pltpu.async_copy(y_hbm.at[src_v.at[b]], msgs_v.at[b], gsem[b])
      plsc.subcore_barrier()

      def step(i, carry):
        for b in range(_NBUF):
          r = i * _NBUF + b
          pltpu.make_async_copy(y_hbm.at[src_v.at[r]], msgs_v.at[b],
                                gsem[b]).wait()
          pltpu.async_copy(msgs_v.at[b], acc_sh.at[dst_v.at[r]], ssem[b],
                           add=True).wait()

          @pl.when(r + _NBUF < _GF)
          def _start_next():
            pltpu.async_copy(y_hbm.at[src_v.at[r + _NBUF]], msgs_v.at[b],
                             gsem[b])
        return carry

      lax.fori_loop(0, _GF // _NBUF, step, 0)
      plsc.subcore_barrier()
      pltpu.sync_copy(acc_sh.at[pl.ds(row0, _RPT)],
                      acc_out.at[pl.ds(row0, _RPT)])

    if with_deg:
      @pl.when(c == 1)
      def _degrees():
        pltpu.sync_copy(zd_hbm, deg_sh.at[pl.ds(row0, _RPT)])
        pltpu.sync_copy(ones_hbm, ones_v)
        pltpu.sync_copy(dst_hbm.at[pl.ds(lo, _GF)], dst_v)
        plsc.subcore_barrier()

        def dstep(i, carry):
          for b in range(_NBUF):
            r = i * _NBUF + b

            @pl.when(r >= _NBUF)
            def _drain():
              pltpu.make_async_copy(ones_v, deg_sh.at[dst_v.at[r]],
                                    dsem[b]).wait()

            pltpu.async_copy(ones_v, deg_sh.at[dst_v.at[r]], dsem[b],
                             add=True)
          return carry

        lax.fori_loop(0, _GF // _NBUF, dstep, 0)
        for b in range(_NBUF):
          pltpu.make_async_copy(ones_v, deg_sh.at[dst_v.at[0]],
                                dsem[b]).wait()
        plsc.subcore_barrier()
        pltpu.sync_copy(deg_sh.at[pl.ds(row0, _RPT)],
                        deg_out.at[pl.ds(row0, _RPT)])

  return pl.kernel(
      body, out_type=out_type, mesh=mesh, scratch_types=scratch,
      compiler_params=pltpu.CompilerParams(use_tc_tiling_on_sc=False))


def _stage_a(x, W1_l, W1_r, b1):
  def body(x_ref, wl_ref, wr_ref, b_ref, y_ref, z_ref):
    xv = x_ref[...]
    y_ref[...] = jnp.dot(xv, wl_ref[...], preferred_element_type=jnp.float32)
    z_ref[...] = (jnp.dot(xv, wr_ref[...], preferred_element_type=jnp.float32)
                  + b_ref[...])

  return pl.pallas_call(
      body,
      out_shape=[jax.ShapeDtypeStruct((_N, _D_HID), jnp.float32),
                 jax.ShapeDtypeStruct((_N, _D_HID), jnp.float32)],
  )(x, W1_l, W1_r, b1)


def _stage_b(acc1, deg, z1, gamma, beta, W2lp, W2_r, b2):
  def body(acc_ref, deg_ref, z1_ref, g_ref, be_ref, wl_ref, wr_ref, b2_ref,
           y2_ref, z2_ref):
    invd = 1.0 / jnp.maximum(deg_ref[:_N, :1], 1.0)
    pre = acc_ref[:_N, :] * invd + z1_ref[...]
    mu = jnp.mean(pre, axis=0, keepdims=True)
    var = jnp.mean((pre - mu) ** 2, axis=0, keepdims=True)
    h = (pre - mu) * lax.rsqrt(var + _EPS) * g_ref[...] + be_ref[...]
    h = jnp.maximum(h, 0.0)
    y2_ref[...] = jnp.dot(h, wl_ref[...], preferred_element_type=jnp.float32)
    z2_ref[...] = (jnp.dot(h, wr_ref[...], preferred_element_type=jnp.float32)
                   + b2_ref[...])

  return pl.pallas_call(
      body,
      out_shape=[jax.ShapeDtypeStruct((_N, _W2P), jnp.float32),
                 jax.ShapeDtypeStruct((_N, _D_OUT), jnp.float32)],
  )(acc1, deg, z1, gamma, beta, W2lp, W2_r, b2)


def _stage_c(acc2, deg, z2, gamma, beta):
  def body(acc_ref, deg_ref, z2_ref, g_ref, be_ref, out_ref):
    invd = 1.0 / jnp.maximum(deg_ref[:_N, :1], 1.0)
    pre = acc_ref[:_N, :_D_OUT] * invd + z2_ref[...]
    mu = jnp.mean(pre, axis=0, keepdims=True)
    var = jnp.mean((pre - mu) ** 2, axis=0, keepdims=True)
    h = (pre - mu) * lax.rsqrt(var + _EPS) * g_ref[...] + be_ref[...]
    m = jnp.max(h, axis=1, keepdims=True)
    lse = jnp.log(jnp.sum(jnp.exp(h - m), axis=1, keepdims=True)) + m
    out_ref[...] = h - lse

  return pl.pallas_call(
      body,
      out_shape=jax.ShapeDtypeStruct((_N, _D_OUT), jnp.float32),
  )(acc2, deg, z2, gamma, beta)


def kernel(x, edge_index, W1_l, W1_r, b1, bn1_gamma, bn1_beta,
           W2_l, W2_r, b2, bn2_gamma, bn2_beta):
  src3 = edge_index[0].reshape(_TG, _GE)
  dst3 = edge_index[1].reshape(_TG, _GE)
  zf32 = jnp.zeros((_RPT, _D_HID), jnp.float32)
  zf16 = jnp.zeros((_RPT, _W2P), jnp.float32)
  zd = jnp.zeros((_RPT, _DW), jnp.float32)
  ones = jnp.ones((_GE, _DW), jnp.float32)
  W2lp = jnp.pad(W2_l, ((0, 0), (0, _W2P - _D_OUT)))

  y1, z1 = _stage_a(x, W1_l, W1_r, b1)
  acc1, deg = _sc_agg(_D_HID, True)(y1, src3, dst3, zf32, zd, ones)
  y2p, z2 = _stage_b(acc1, deg, z1, bn1_gamma, bn1_beta, W2lp, W2_r, b2)
  (acc2,) = _sc_agg(_W2P, False)(y2p, src3, dst3, zf16)
  return _stage_c(acc2, deg, z2, bn2_gamma, bn2_beta)


# speedup vs baseline: 1.8788x; 1.1478x over previous
import jax
import jax.numpy as jnp
from jax import lax
from jax.experimental import pallas as pl
from jax.experimental.pallas import tpu as pltpu
from jax.experimental.pallas import tpu_sc as plsc

_N = 10000
_E = 320000
_D_IN = 128
_D_HID = 32
_D_OUT = 2
_W2P = 16
_EPS = 1e-5

_NC = 2
_NS = 16
_NW = _NC * _NS
_GE = 500
_TG = 640
_GF = _TG // _NW
_NBUF = 2
_ROWS = 10112
_RPT = _ROWS // _NS
_DW = 8


def _sc_agg(width, with_deg):
  mesh = plsc.VectorSubcoreMesh(core_axis_name="c", subcore_axis_name="s",
                                num_cores=_NC, num_subcores=_NS)
  out_type = [jax.ShapeDtypeStruct((_NC * _ROWS, width), jnp.float32)]
  scratch = [
      pltpu.VMEM((_GF, _GE), jnp.int32),
      pltpu.VMEM((_GF, _GE), jnp.int32),
      pltpu.VMEM((_NBUF, _GE, width), jnp.float32),
      pltpu.VMEM_SHARED((_ROWS, width), jnp.float32),
  ]
  if with_deg:
    out_type.append(jax.ShapeDtypeStruct((_NC * _ROWS, _DW), jnp.float32))
    scratch += [
        pltpu.VMEM((_GE, _DW), jnp.float32),
        pltpu.VMEM_SHARED((_ROWS, _DW), jnp.float32),
    ]
  scratch += [pltpu.SemaphoreType.DMA] * (3 * _NBUF)

  def body(*refs):
    if with_deg:
      (y_hbm, e_hbm, zf_hbm, zd_hbm, ones_hbm, acc_out, deg_out,
       src_v, dst_v, msgs_v, acc_sh, ones_v, deg_sh, *sems) = refs
    else:
      (y_hbm, e_hbm, zf_hbm, acc_out,
       src_v, dst_v, msgs_v, acc_sh, *sems) = refs
    gsem = sems[:_NBUF]
    ssem = sems[_NBUF:2 * _NBUF]
    dsem = sems[2 * _NBUF:]

    c = lax.axis_index("c")
    s = lax.axis_index("s")
    row0 = s * _RPT
    lo = (c * _NS + s) * _GF

    pltpu.sync_copy(zf_hbm, acc_sh.at[pl.ds(row0, _RPT)])
    if with_deg:
      pltpu.sync_copy(zd_hbm, deg_sh.at[pl.ds(row0, _RPT)])
      pltpu.sync_copy(ones_hbm, ones_v)
    pltpu.sync_copy(e_hbm.at[0, pl.ds(lo, _GF)], src_v)
    pltpu.sync_copy(e_hbm.at[1, pl.ds(lo, _GF)], dst_v)
    for b in range(_NBUF):
      pltpu.async_copy(y_hbm.at[src_v.at[b]], msgs_v.at[b], gsem[b])
    plsc.subcore_barrier()

    def step(i, carry):
      for b in range(_NBUF):
        r = i * _NBUF + b
        pltpu.make_async_copy(y_hbm.at[src_v.at[r]], msgs_v.at[b],
                              gsem[b]).wait()
        sd = pltpu.async_copy(msgs_v.at[b], acc_sh.at[dst_v.at[r]], ssem[b],
                              add=True)
        if with_deg:
          @pl.when(r >= _NBUF)
          def _drain():
            pltpu.make_async_copy(ones_v, deg_sh.at[dst_v.at[r]],
                                  dsem[b]).wait()

          pltpu.async_copy(ones_v, deg_sh.at[dst_v.at[r]], dsem[b], add=True)
        sd.wait()

        @pl.when(r + _NBUF < _GF)
        def _start_next():
          pltpu.async_copy(y_hbm.at[src_v.at[r + _NBUF]], msgs_v.at[b],
                           gsem[b])
      return carry

    lax.fori_loop(0, _GF // _NBUF, step, 0)
    if with_deg:
      for b in range(_NBUF):
        pltpu.make_async_copy(ones_v, deg_sh.at[dst_v.at[0]], dsem[b]).wait()
    plsc.subcore_barrier()
    out0 = c * _ROWS + row0
    pltpu.sync_copy(acc_sh.at[pl.ds(row0, _RPT)],
                    acc_out.at[pl.ds(out0, _RPT)])
    if with_deg:
      pltpu.sync_copy(deg_sh.at[pl.ds(row0, _RPT)],
                      deg_out.at[pl.ds(out0, _RPT)])

  return pl.kernel(
      body, out_type=out_type, mesh=mesh, scratch_types=scratch,
      compiler_params=pltpu.CompilerParams(use_tc_tiling_on_sc=False))


def _stage_a(x, W1_l, W1_r, b1):
  def body(x_ref, wl_ref, wr_ref, b_ref, y_ref, z_ref):
    xv = x_ref[...]
    y_ref[...] = jnp.dot(xv, wl_ref[...], preferred_element_type=jnp.float32)
    z_ref[...] = (jnp.dot(xv, wr_ref[...], preferred_element_type=jnp.float32)
                  + b_ref[...])

  return pl.pallas_call(
      body,
      out_shape=[jax.ShapeDtypeStruct((_N, _D_HID), jnp.float32),
                 jax.ShapeDtypeStruct((_N, _D_HID), jnp.float32)],
  )(x, W1_l, W1_r, b1)


def _stage_b(acc1, deg, z1, gamma, beta, W2lp, W2_r, b2):
  def body(acc_ref, deg_ref, z1_ref, g_ref, be_ref, wl_ref, wr_ref, b2_ref,
           y2_ref, z2_ref):
    dsum = deg_ref[0, :_N, :1] + deg_ref[1, :_N, :1]
    invd = 1.0 / jnp.maximum(dsum, 1.0)
    pre = (acc_ref[0, :_N, :] + acc_ref[1, :_N, :]) * invd + z1_ref[...]
    mu = jnp.mean(pre, axis=0, keepdims=True)
    var = jnp.mean((pre - mu) ** 2, axis=0, keepdims=True)
    h = (pre - mu) * lax.rsqrt(var + _EPS) * g_ref[...] + be_ref[...]
    h = jnp.maximum(h, 0.0)
    y2_ref[...] = jnp.dot(h, wl_ref[...], preferred_element_type=jnp.float32)
    z2_ref[...] = (jnp.dot(h, wr_ref[...], preferred_element_type=jnp.float32)
                   + b2_ref[...])

  return pl.pallas_call(
      body,
      out_shape=[jax.ShapeDtypeStruct((_N, _W2P), jnp.float32),
                 jax.ShapeDtypeStruct((_N, _D_OUT), jnp.float32)],
  )(acc1, deg, z1, gamma, beta, W2lp, W2_r, b2)


def _stage_c(acc2, deg, z2, gamma, beta):
  def body(acc_ref, deg_ref, z2_ref, g_ref, be_ref, out_ref):
    dsum = deg_ref[0, :_N, :1] + deg_ref[1, :_N, :1]
    invd = 1.0 / jnp.maximum(dsum, 1.0)
    pre = (acc_ref[0, :_N, :_D_OUT] + acc_ref[1, :_N, :_D_OUT]) * invd \
        + z2_ref[...]
    mu = jnp.mean(pre, axis=0, keepdims=True)
    var = jnp.mean((pre - mu) ** 2, axis=0, keepdims=True)
    h = (pre - mu) * lax.rsqrt(var + _EPS) * g_ref[...] + be_ref[...]
    m = jnp.max(h, axis=1, keepdims=True)
    lse = jnp.log(jnp.sum(jnp.exp(h - m), axis=1, keepdims=True)) + m
    out_ref[...] = h - lse

  return pl.pallas_call(
      body,
      out_shape=jax.ShapeDtypeStruct((_N, _D_OUT), jnp.float32),
  )(acc2, deg, z2, gamma, beta)


def kernel(x, edge_index, W1_l, W1_r, b1, bn1_gamma, bn1_beta,
           W2_l, W2_r, b2, bn2_gamma, bn2_beta):
  e3 = edge_index.reshape(2, _TG, _GE)
  zf32 = jnp.zeros((_RPT, _D_HID), jnp.float32)
  zf16 = jnp.zeros((_RPT, _W2P), jnp.float32)
  zd = jnp.zeros((_RPT, _DW), jnp.float32)
  ones = jnp.ones((_GE, _DW), jnp.float32)
  W2lp = jnp.pad(W2_l, ((0, 0), (0, _W2P - _D_OUT)))

  y1, z1 = _stage_a(x, W1_l, W1_r, b1)
  acc1, deg = _sc_agg(_D_HID, True)(y1, e3, zf32, zd, ones)
  acc1 = acc1.reshape(_NC, _ROWS, _D_HID)
  deg3 = deg.reshape(_NC, _ROWS, _DW)
  y2p, z2 = _stage_b(acc1, deg3, z1, bn1_gamma, bn1_beta, W2lp, W2_r, b2)
  (acc2,) = _sc_agg(_W2P, False)(y2p, e3, zf16)
  acc2 = acc2.reshape(_NC, _ROWS, _W2P)
  return _stage_c(acc2, deg3, z2, bn2_gamma, bn2_beta)


# speedup vs baseline: 1.8887x; 1.0053x over previous
import jax
import jax.numpy as jnp
from jax import lax
from jax.experimental import pallas as pl
from jax.experimental.pallas import tpu as pltpu
from jax.experimental.pallas import tpu_sc as plsc

_N = 10000
_E = 320000
_D_IN = 128
_D_HID = 32
_D_OUT = 2
_W2P = 16
_EPS = 1e-5

_NC = 2
_NS = 16
_NW = _NC * _NS
_GE = 500
_TG = 640
_GF = _TG // _NW
_NBUF = 2
_ROWS = 10112
_RPT = _ROWS // _NS
_DW = 8


def _sc_agg(width, with_deg):
  mesh = plsc.VectorSubcoreMesh(core_axis_name="c", subcore_axis_name="s",
                                num_cores=_NC, num_subcores=_NS)
  out_type = [jax.ShapeDtypeStruct((_NC, _ROWS, width), jnp.float32)]
  scratch = [
      pltpu.VMEM((_GF, _GE), jnp.int32),
      pltpu.VMEM((_GF, _GE), jnp.int32),
      pltpu.VMEM((_NBUF, _GE, width), jnp.float32),
      pltpu.VMEM_SHARED((_ROWS, width), jnp.float32),
  ]
  if with_deg:
    out_type.append(jax.ShapeDtypeStruct((_NC, _ROWS, _DW), jnp.float32))
    scratch += [
        pltpu.VMEM((_GE, _DW), jnp.float32),
        pltpu.VMEM_SHARED((_ROWS, _DW), jnp.float32),
    ]
  scratch += [pltpu.SemaphoreType.DMA] * (3 * _NBUF)

  def body(*refs):
    if with_deg:
      (y_hbm, e_hbm, zf_hbm, zd_hbm, ones_hbm, acc_out, deg_out,
       src_v, dst_v, msgs_v, acc_sh, ones_v, deg_sh, *sems) = refs
    else:
      (y_hbm, e_hbm, zf_hbm, acc_out,
       src_v, dst_v, msgs_v, acc_sh, *sems) = refs
    gsem = sems[:_NBUF]
    ssem = sems[_NBUF:2 * _NBUF]
    dsem = sems[2 * _NBUF:]

    c = lax.axis_index("c")
    s = lax.axis_index("s")
    row0 = s * _RPT
    lo = (c * _NS + s) * _GF

    pltpu.sync_copy(zf_hbm, acc_sh.at[pl.ds(row0, _RPT)])
    if with_deg:
      pltpu.sync_copy(zd_hbm, deg_sh.at[pl.ds(row0, _RPT)])
      pltpu.sync_copy(ones_hbm, ones_v)
    pltpu.sync_copy(e_hbm.at[0, pl.ds(lo, _GF)], src_v)
    pltpu.sync_copy(e_hbm.at[1, pl.ds(lo, _GF)], dst_v)
    for b in range(_NBUF):
      pltpu.async_copy(y_hbm.at[src_v.at[b]], msgs_v.at[b], gsem[b])
    plsc.subcore_barrier()

    def step(i, carry):
      for b in range(_NBUF):
        r = i * _NBUF + b
        pltpu.make_async_copy(y_hbm.at[src_v.at[r]], msgs_v.at[b],
                              gsem[b]).wait()
        sd = pltpu.async_copy(msgs_v.at[b], acc_sh.at[dst_v.at[r]], ssem[b],
                              add=True)
        if with_deg:
          @pl.when(r >= _NBUF)
          def _drain():
            pltpu.make_async_copy(ones_v, deg_sh.at[dst_v.at[r]],
                                  dsem[b]).wait()

          pltpu.async_copy(ones_v, deg_sh.at[dst_v.at[r]], dsem[b], add=True)
        sd.wait()

        @pl.when(r + _NBUF < _GF)
        def _start_next():
          pltpu.async_copy(y_hbm.at[src_v.at[r + _NBUF]], msgs_v.at[b],
                           gsem[b])
      return carry

    lax.fori_loop(0, _GF // _NBUF, step, 0)
    if with_deg:
      for b in range(_NBUF):
        pltpu.make_async_copy(ones_v, deg_sh.at[dst_v.at[0]], dsem[b]).wait()
    plsc.subcore_barrier()
    pltpu.sync_copy(acc_sh.at[pl.ds(row0, _RPT)],
                    acc_out.at[c, pl.ds(row0, _RPT)])
    if with_deg:
      pltpu.sync_copy(deg_sh.at[pl.ds(row0, _RPT)],
                      deg_out.at[c, pl.ds(row0, _RPT)])

  return pl.kernel(
      body, out_type=out_type, mesh=mesh, scratch_types=scratch,
      compiler_params=pltpu.CompilerParams(use_tc_tiling_on_sc=False))


def _stage_a(x, W1_l, W1_r, b1):
  def body(x_ref, wl_ref, wr_ref, b_ref, y_ref, z_ref):
    xv = x_ref[...]
    y_ref[...] = jnp.dot(xv, wl_ref[...], preferred_element_type=jnp.float32)
    z_ref[...] = (jnp.dot(xv, wr_ref[...], preferred_element_type=jnp.float32)
                  + b_ref[...])

  return pl.pallas_call(
      body,
      out_shape=[jax.ShapeDtypeStruct((_N, _D_HID), jnp.float32),
                 jax.ShapeDtypeStruct((_N, _D_HID), jnp.float32)],
  )(x, W1_l, W1_r, b1)


def _stage_b(acc1, deg, z1, gamma, beta, W2lp, W2_r, b2):
  def body(acc_ref, deg_ref, z1_ref, g_ref, be_ref, wl_ref, wr_ref, b2_ref,
           y2_ref, z2_ref):
    dsum = deg_ref[0, :_N, :1] + deg_ref[1, :_N, :1]
    invd = 1.0 / jnp.maximum(dsum, 1.0)
    pre = (acc_ref[0, :_N, :] + acc_ref[1, :_N, :]) * invd + z1_ref[...]
    mu = jnp.mean(pre, axis=0, keepdims=True)
    var = jnp.mean((pre - mu) ** 2, axis=0, keepdims=True)
    h = (pre - mu) * lax.rsqrt(var + _EPS) * g_ref[...] + be_ref[...]
    h = jnp.maximum(h, 0.0)
    y2_ref[...] = jnp.dot(h, wl_ref[...], preferred_element_type=jnp.float32)
    z2_ref[...] = (jnp.dot(h, wr_ref[...], preferred_element_type=jnp.float32)
                   + b2_ref[...])

  return pl.pallas_call(
      body,
      out_shape=[jax.ShapeDtypeStruct((_N, _W2P), jnp.float32),
                 jax.ShapeDtypeStruct((_N, _D_OUT), jnp.float32)],
  )(acc1, deg, z1, gamma, beta, W2lp, W2_r, b2)


def _stage_c(acc2, deg, z2, gamma, beta):
  def body(acc_ref, deg_ref, z2_ref, g_ref, be_ref, out_ref):
    dsum = deg_ref[0, :_N, :1] + deg_ref[1, :_N, :1]
    invd = 1.0 / jnp.maximum(dsum, 1.0)
    pre = (acc_ref[0, :_N, :_D_OUT] + acc_ref[1, :_N, :_D_OUT]) * invd \
        + z2_ref[...]
    mu = jnp.mean(pre, axis=0, keepdims=True)
    var = jnp.mean((pre - mu) ** 2, axis=0, keepdims=True)
    h = (pre - mu) * lax.rsqrt(var + _EPS) * g_ref[...] + be_ref[...]
    m = jnp.max(h, axis=1, keepdims=True)
    lse = jnp.log(jnp.sum(jnp.exp(h - m), axis=1, keepdims=True)) + m
    out_ref[...] = h - lse

  return pl.pallas_call(
      body,
      out_shape=jax.ShapeDtypeStruct((_N, _D_OUT), jnp.float32),
  )(acc2, deg, z2, gamma, beta)


def kernel(x, edge_index, W1_l, W1_r, b1, bn1_gamma, bn1_beta,
           W2_l, W2_r, b2, bn2_gamma, bn2_beta):
  e3 = edge_index.reshape(2, _TG, _GE)
  zf32 = jnp.zeros((_RPT, _D_HID), jnp.float32)
  zf16 = jnp.zeros((_RPT, _W2P), jnp.float32)
  zd = jnp.zeros((_RPT, _DW), jnp.float32)
  ones = jnp.ones((_GE, _DW), jnp.float32)
  W2lp = jnp.pad(W2_l, ((0, 0), (0, _W2P - _D_OUT)))

  y1, z1 = _stage_a(x, W1_l, W1_r, b1)
  acc1, deg = _sc_agg(_D_HID, True)(y1, e3, zf32, zd, ones)
  y2p, z2 = _stage_b(acc1, deg, z1, bn1_gamma, bn1_beta, W2lp, W2_r, b2)
  (acc2,) = _sc_agg(_W2P, False)(y2p, e3, zf16)
  return _stage_c(acc2, deg, z2, bn2_gamma, bn2_beta)


# speedup vs baseline: 1.9238x; 1.0186x over previous
import jax
import jax.numpy as jnp
from jax import lax
from jax.experimental import pallas as pl
from jax.experimental.pallas import tpu as pltpu
from jax.experimental.pallas import tpu_sc as plsc

_N = 10000
_E = 320000
_D_IN = 128
_D_HID = 32
_D_OUT = 2
_W2P = 16
_EPS = 1e-5

_NC = 2
_NS = 16
_NW = _NC * _NS
_GE = 500
_TG = 640
_GF = _TG // _NW
_NBUF = 2
_ROWS = 10112
_RPT = _ROWS // _NS
_DW = 8


def _sc_agg(width, with_deg):
  mesh = plsc.VectorSubcoreMesh(core_axis_name="c", subcore_axis_name="s",
                                num_cores=_NC, num_subcores=_NS)
  out_type = [jax.ShapeDtypeStruct((_NC, _ROWS, width), jnp.float32)]
  scratch = [
      pltpu.VMEM((_GF, _GE), jnp.int32),
      pltpu.VMEM((_GF, _GE), jnp.int32),
      pltpu.VMEM((_NBUF, _GE, width), jnp.float32),
      pltpu.VMEM_SHARED((_ROWS, width), jnp.float32),
  ]
  if with_deg:
    out_type.append(jax.ShapeDtypeStruct((_NC, _ROWS, _DW), jnp.float32))
    scratch += [
        pltpu.VMEM((_GE, _DW), jnp.float32),
        pltpu.VMEM_SHARED((_ROWS, _DW), jnp.float32),
    ]
  scratch += [pltpu.SemaphoreType.DMA] * (3 * _NBUF)

  def body(*refs):
    if with_deg:
      (y_hbm, e_hbm, zf_hbm, zd_hbm, ones_hbm, acc_out, deg_out,
       src_v, dst_v, msgs_v, acc_sh, ones_v, deg_sh, *sems) = refs
    else:
      (y_hbm, e_hbm, zf_hbm, acc_out,
       src_v, dst_v, msgs_v, acc_sh, *sems) = refs
    gsem = sems[:_NBUF]
    ssem = sems[_NBUF:2 * _NBUF]
    dsem = sems[2 * _NBUF:]

    c = lax.axis_index("c")
    s = lax.axis_index("s")
    row0 = s * _RPT
    lo = (c * _NS + s) * _GF

    pltpu.sync_copy(zf_hbm, acc_sh.at[pl.ds(row0, _RPT)])
    if with_deg:
      pltpu.sync_copy(zd_hbm, deg_sh.at[pl.ds(row0, _RPT)])
      pltpu.sync_copy(ones_hbm, ones_v)
    pltpu.sync_copy(e_hbm.at[0, pl.ds(lo, _GF)], src_v)
    pltpu.sync_copy(e_hbm.at[1, pl.ds(lo, _GF)], dst_v)
    for b in range(_NBUF):
      pltpu.async_copy(y_hbm.at[src_v.at[b]], msgs_v.at[b], gsem[b])
    plsc.subcore_barrier()

    def step(i, carry):
      for b in range(_NBUF):
        r = i * _NBUF + b
        pltpu.make_async_copy(y_hbm.at[src_v.at[r]], msgs_v.at[b],
                              gsem[b]).wait()
        sd = pltpu.async_copy(msgs_v.at[b], acc_sh.at[dst_v.at[r]], ssem[b],
                              add=True)
        if with_deg:
          @pl.when(r >= _NBUF)
          def _drain():
            pltpu.make_async_copy(ones_v, deg_sh.at[dst_v.at[r]],
                                  dsem[b]).wait()

          pltpu.async_copy(ones_v, deg_sh.at[dst_v.at[r]], dsem[b], add=True)
        sd.wait()

        @pl.when(r + _NBUF < _GF)
        def _start_next():
          pltpu.async_copy(y_hbm.at[src_v.at[r + _NBUF]], msgs_v.at[b],
                           gsem[b])
      return carry

    lax.fori_loop(0, _GF // _NBUF, step, 0)
    if with_deg:
      for b in range(_NBUF):
        pltpu.make_async_copy(ones_v, deg_sh.at[dst_v.at[0]], dsem[b]).wait()
    plsc.subcore_barrier()
    pltpu.sync_copy(acc_sh.at[pl.ds(row0, _RPT)],
                    acc_out.at[c, pl.ds(row0, _RPT)])
    if with_deg:
      pltpu.sync_copy(deg_sh.at[pl.ds(row0, _RPT)],
                      deg_out.at[c, pl.ds(row0, _RPT)])

  return pl.kernel(
      body, out_type=out_type, mesh=mesh, scratch_types=scratch,
      compiler_params=pltpu.CompilerParams(use_tc_tiling_on_sc=False))


def _stage_a(x, W1_l, W1_r, b1):
  def body(x_ref, wl_ref, wr_ref, b_ref, y_ref, z_ref):
    xv = x_ref[...]
    y_ref[...] = jnp.dot(xv, wl_ref[...], preferred_element_type=jnp.float32)
    z_ref[...] = (jnp.dot(xv, wr_ref[...], preferred_element_type=jnp.float32)
                  + b_ref[...])

  return pl.pallas_call(
      body,
      out_shape=[jax.ShapeDtypeStruct((_N, _D_HID), jnp.float32),
                 jax.ShapeDtypeStruct((_N, _D_HID), jnp.float32)],
  )(x, W1_l, W1_r, b1)


def _stage_b(acc1, deg, z1, gamma, beta, W2lp, W2_r, b2):
  def body(acc_ref, deg_ref, z1_ref, g_ref, be_ref, wl_ref, wr_ref, b2_ref,
           y2_ref, z2_ref):
    dsum = deg_ref[0, :_N, :1] + deg_ref[1, :_N, :1]
    invd = 1.0 / jnp.maximum(dsum, 1.0)
    pre = (acc_ref[0, :_N, :] + acc_ref[1, :_N, :]) * invd + z1_ref[...]
    mu = jnp.mean(pre, axis=0, keepdims=True)
    var = jnp.mean((pre - mu) ** 2, axis=0, keepdims=True)
    h = (pre - mu) * lax.rsqrt(var + _EPS) * g_ref[...] + be_ref[...]
    h = jnp.maximum(h, 0.0)
    y2_ref[...] = jnp.dot(h, wl_ref[...], preferred_element_type=jnp.float32)
    z2 = (jnp.dot(h, wr_ref[...], preferred_element_type=jnp.float32)
          + b2_ref[...])
    z2_ref[...] = jnp.concatenate(
        [z2, invd, jnp.zeros((_N, _W2P - _D_OUT - 1), jnp.float32)], axis=1)

  return pl.pallas_call(
      body,
      out_shape=[jax.ShapeDtypeStruct((_N, _W2P), jnp.float32),
                 jax.ShapeDtypeStruct((_N, _W2P), jnp.float32)],
  )(acc1, deg, z1, gamma, beta, W2lp, W2_r, b2)


def _stage_c(acc2, z2e, gamma, beta):
  def body(acc_ref, z2_ref, g_ref, be_ref, out_ref):
    invd = z2_ref[:, _D_OUT:_D_OUT + 1]
    pre = (acc_ref[0, :_N, :_D_OUT] + acc_ref[1, :_N, :_D_OUT]) * invd \
        + z2_ref[:, :_D_OUT]
    mu = jnp.mean(pre, axis=0, keepdims=True)
    var = jnp.mean((pre - mu) ** 2, axis=0, keepdims=True)
    h = (pre - mu) * lax.rsqrt(var + _EPS) * g_ref[...] + be_ref[...]
    m = jnp.max(h, axis=1, keepdims=True)
    lse = jnp.log(jnp.sum(jnp.exp(h - m), axis=1, keepdims=True)) + m
    out_ref[...] = h - lse

  return pl.pallas_call(
      body,
      out_shape=jax.ShapeDtypeStruct((_N, _D_OUT), jnp.float32),
  )(acc2, z2e, gamma, beta)


def kernel(x, edge_index, W1_l, W1_r, b1, bn1_gamma, bn1_beta,
           W2_l, W2_r, b2, bn2_gamma, bn2_beta):
  e3 = edge_index.reshape(2, _TG, _GE)
  zf32 = jnp.zeros((_RPT, _D_HID), jnp.float32)
  zf16 = jnp.zeros((_RPT, _W2P), jnp.float32)
  zd = jnp.zeros((_RPT, _DW), jnp.float32)
  ones = jnp.ones((_GE, _DW), jnp.float32)
  W2lp = jnp.pad(W2_l, ((0, 0), (0, _W2P - _D_OUT)))

  y1, z1 = _stage_a(x, W1_l, W1_r, b1)
  acc1, deg = _sc_agg(_D_HID, True)(y1, e3, zf32, zd, ones)
  y2p, z2e = _stage_b(acc1, deg, z1, bn1_gamma, bn1_beta, W2lp, W2_r, b2)
  (acc2,) = _sc_agg(_W2P, False)(y2p, e3, zf16)
  return _stage_c(acc2, z2e, bn2_gamma, bn2_beta)


# speedup vs baseline: 2.0027x; 1.0410x over previous
import jax
import jax.numpy as jnp
from jax import lax
from jax.experimental import pallas as pl
from jax.experimental.pallas import tpu as pltpu
from jax.experimental.pallas import tpu_sc as plsc

_N = 10000
_E = 320000
_D_IN = 128
_D_HID = 32
_D_OUT = 2
_W2P = 16
_EPS = 1e-5

_NC = 2
_NS = 16
_NW = _NC * _NS
_GE = 250
_TG = 1280
_GF = _TG // _NW
_NBUF = 4
_ROWS = 10112
_RPT = _ROWS // _NS
_DW = 8


def _sc_agg(width, with_deg):
  mesh = plsc.VectorSubcoreMesh(core_axis_name="c", subcore_axis_name="s",
                                num_cores=_NC, num_subcores=_NS)
  out_type = [jax.ShapeDtypeStruct((_NC, _ROWS, width), jnp.float32)]
  scratch = [
      pltpu.VMEM((_GF, _GE), jnp.int32),
      pltpu.VMEM((_GF, _GE), jnp.int32),
      pltpu.VMEM((_NBUF, _GE, width), jnp.float32),
      pltpu.VMEM_SHARED((_ROWS, width), jnp.float32),
  ]
  if with_deg:
    out_type.append(jax.ShapeDtypeStruct((_NC, _ROWS, _DW), jnp.float32))
    scratch += [
        pltpu.VMEM((_GE, _DW), jnp.float32),
        pltpu.VMEM_SHARED((_ROWS, _DW), jnp.float32),
    ]
  scratch += [pltpu.SemaphoreType.DMA] * (3 * _NBUF)

  def body(*refs):
    if with_deg:
      (y_hbm, e_hbm, zf_hbm, zd_hbm, ones_hbm, acc_out, deg_out,
       src_v, dst_v, msgs_v, acc_sh, ones_v, deg_sh, *sems) = refs
    else:
      (y_hbm, e_hbm, zf_hbm, acc_out,
       src_v, dst_v, msgs_v, acc_sh, *sems) = refs
    gsem = sems[:_NBUF]
    ssem = sems[_NBUF:2 * _NBUF]
    dsem = sems[2 * _NBUF:]

    c = lax.axis_index("c")
    s = lax.axis_index("s")
    row0 = s * _RPT
    lo = (c * _NS + s) * _GF

    pltpu.sync_copy(zf_hbm, acc_sh.at[pl.ds(row0, _RPT)])
    if with_deg:
      pltpu.sync_copy(zd_hbm, deg_sh.at[pl.ds(row0, _RPT)])
      pltpu.sync_copy(ones_hbm, ones_v)
    pltpu.sync_copy(e_hbm.at[0, pl.ds(lo, _GF)], src_v)
    pltpu.sync_copy(e_hbm.at[1, pl.ds(lo, _GF)], dst_v)
    for b in range(_NBUF):
      pltpu.async_copy(y_hbm.at[src_v.at[b]], msgs_v.at[b], gsem[b])
    plsc.subcore_barrier()

    def step(i, carry):
      for b in range(_NBUF):
        r = i * _NBUF + b
        pltpu.make_async_copy(y_hbm.at[src_v.at[r]], msgs_v.at[b],
                              gsem[b]).wait()
        sd = pltpu.async_copy(msgs_v.at[b], acc_sh.at[dst_v.at[r]], ssem[b],
                              add=True)
        if with_deg:
          @pl.when(r >= _NBUF)
          def _drain():
            pltpu.make_async_copy(ones_v, deg_sh.at[dst_v.at[r]],
                                  dsem[b]).wait()

          pltpu.async_copy(ones_v, deg_sh.at[dst_v.at[r]], dsem[b], add=True)
        sd.wait()

        @pl.when(r + _NBUF < _GF)
        def _start_next():
          pltpu.async_copy(y_hbm.at[src_v.at[r + _NBUF]], msgs_v.at[b],
                           gsem[b])
      return carry

    lax.fori_loop(0, _GF // _NBUF, step, 0)
    if with_deg:
      for b in range(_NBUF):
        pltpu.make_async_copy(ones_v, deg_sh.at[dst_v.at[0]], dsem[b]).wait()
    plsc.subcore_barrier()
    pltpu.sync_copy(acc_sh.at[pl.ds(row0, _RPT)],
                    acc_out.at[c, pl.ds(row0, _RPT)])
    if with_deg:
      pltpu.sync_copy(deg_sh.at[pl.ds(row0, _RPT)],
                      deg_out.at[c, pl.ds(row0, _RPT)])

  return pl.kernel(
      body, out_type=out_type, mesh=mesh, scratch_types=scratch,
      compiler_params=pltpu.CompilerParams(use_tc_tiling_on_sc=False))


def _stage_a(x, W1_l, W1_r, b1):
  def body(x_ref, wl_ref, wr_ref, b_ref, y_ref, z_ref):
    xv = x_ref[...]
    y_ref[...] = jnp.dot(xv, wl_ref[...], preferred_element_type=jnp.float32)
    z_ref[...] = (jnp.dot(xv, wr_ref[...], preferred_element_type=jnp.float32)
                  + b_ref[...])

  return pl.pallas_call(
      body,
      out_shape=[jax.ShapeDtypeStruct((_N, _D_HID), jnp.float32),
                 jax.ShapeDtypeStruct((_N, _D_HID), jnp.float32)],
  )(x, W1_l, W1_r, b1)


def _stage_b(acc1, deg, z1, gamma, beta, W2lp, W2_r, b2):
  def body(acc_ref, deg_ref, z1_ref, g_ref, be_ref, wl_ref, wr_ref, b2_ref,
           y2_ref, z2_ref):
    dsum = deg_ref[0, :_N, :1] + deg_ref[1, :_N, :1]
    invd = 1.0 / jnp.maximum(dsum, 1.0)
    pre = (acc_ref[0, :_N, :] + acc_ref[1, :_N, :]) * invd + z1_ref[...]
    mu = jnp.mean(pre, axis=0, keepdims=True)
    var = jnp.mean((pre - mu) ** 2, axis=0, keepdims=True)
    h = (pre - mu) * lax.rsqrt(var + _EPS) * g_ref[...] + be_ref[...]
    h = jnp.maximum(h, 0.0)
    y2_ref[...] = jnp.dot(h, wl_ref[...], preferred_element_type=jnp.float32)
    z2 = (jnp.dot(h, wr_ref[...], preferred_element_type=jnp.float32)
          + b2_ref[...])
    z2_ref[...] = jnp.concatenate(
        [z2, invd, jnp.zeros((_N, _W2P - _D_OUT - 1), jnp.float32)], axis=1)

  return pl.pallas_call(
      body,
      out_shape=[jax.ShapeDtypeStruct((_N, _W2P), jnp.float32),
                 jax.ShapeDtypeStruct((_N, _W2P), jnp.float32)],
  )(acc1, deg, z1, gamma, beta, W2lp, W2_r, b2)


def _stage_c(acc2, z2e, gamma, beta):
  def body(acc_ref, z2_ref, g_ref, be_ref, out_ref):
    invd = z2_ref[:, _D_OUT:_D_OUT + 1]
    pre = (acc_ref[0, :_N, :_D_OUT] + acc_ref[1, :_N, :_D_OUT]) * invd \
        + z2_ref[:, :_D_OUT]
    mu = jnp.mean(pre, axis=0, keepdims=True)
    var = jnp.mean((pre - mu) ** 2, axis=0, keepdims=True)
    h = (pre - mu) * lax.rsqrt(var + _EPS) * g_ref[...] + be_ref[...]
    m = jnp.max(h, axis=1, keepdims=True)
    lse = jnp.log(jnp.sum(jnp.exp(h - m), axis=1, keepdims=True)) + m
    out_ref[...] = h - lse

  return pl.pallas_call(
      body,
      out_shape=jax.ShapeDtypeStruct((_N, _D_OUT), jnp.float32),
  )(acc2, z2e, gamma, beta)


def kernel(x, edge_index, W1_l, W1_r, b1, bn1_gamma, bn1_beta,
           W2_l, W2_r, b2, bn2_gamma, bn2_beta):
  e3 = edge_index.reshape(2, _TG, _GE)
  zf32 = jnp.zeros((_RPT, _D_HID), jnp.float32)
  zf16 = jnp.zeros((_RPT, _W2P), jnp.float32)
  zd = jnp.zeros((_RPT, _DW), jnp.float32)
  ones = jnp.ones((_GE, _DW), jnp.float32)
  W2lp = jnp.pad(W2_l, ((0, 0), (0, _W2P - _D_OUT)))

  y1, z1 = _stage_a(x, W1_l, W1_r, b1)
  acc1, deg = _sc_agg(_D_HID, True)(y1, e3, zf32, zd, ones)
  y2p, z2e = _stage_b(acc1, deg, z1, bn1_gamma, bn1_beta, W2lp, W2_r, b2)
  (acc2,) = _sc_agg(_W2P, False)(y2p, e3, zf16)
  return _stage_c(acc2, z2e, bn2_gamma, bn2_beta)
